# initial kernel scaffold (unmeasured)
import jax
import jax.numpy as jnp
from jax import lax
from jax.experimental import pallas as pl
from jax.experimental.pallas import tpu as pltpu

N_DEV = 16

_RING = [0, 4, 8, 12, 13, 9, 5, 1, 2, 6, 10, 14, 15, 11, 7, 3]
_INV = [0] * N_DEV
for _r, _k in enumerate(_RING):
    _INV[_k] = _r
_NEXT = [_RING[(_INV[k] + 1) % N_DEV] for k in range(N_DEV)]
_PREV = [_RING[(_INV[k] - 1) % N_DEV] for k in range(N_DEV)]
_ORIGIN = [
    [_RING[(_INV[k] - (h + 1)) % N_DEV] for k in range(N_DEV)]
    for h in range(N_DEV - 1)
]


def _gelu(y):
    c = 0.7978845608028654
    return 0.5 * y * (1.0 + jnp.tanh(c * (y + 0.044715 * y * y * y)))


def kernel(x, w_mat):
    m_per, k_dim = x.shape
    _, n_per = w_mat.shape

    my = lax.axis_index("i")
    nxt = jnp.asarray(_NEXT, dtype=jnp.int32)[my]
    prv = jnp.asarray(_PREV, dtype=jnp.int32)[my]
    origins = jnp.asarray(_ORIGIN, dtype=jnp.int32)[:, my]
    tbl = jnp.concatenate(
        [jnp.stack([my.astype(jnp.int32), nxt, prv]), origins]
    )

    def body(tbl_ref, x_ref, w_ref, out_ref, comm_ref, send_sems, recv_sems):
        my_pos = tbl_ref[0]
        right = tbl_ref[1]
        left = tbl_ref[2]

        barrier = pltpu.get_barrier_semaphore()
        for nbr in (left, right):
            pl.semaphore_signal(
                barrier, inc=1, device_id=(nbr,),
                device_id_type=pl.DeviceIdType.MESH,
            )
        pl.semaphore_wait(barrier, 2)

        comm_ref[0, :, :] = x_ref[:, :]

        for h in range(N_DEV - 1):
            s = h % 2
            r = (h + 1) % 2
            rdma = pltpu.make_async_remote_copy(
                src_ref=comm_ref.at[s],
                dst_ref=comm_ref.at[r],
                send_sem=send_sems.at[s],
                recv_sem=recv_sems.at[r],
                device_id=(right,),
                device_id_type=pl.DeviceIdType.MESH,
            )
            rdma.start()
            if h == 0:
                y = jnp.dot(
                    x_ref[:, :], w_ref[:, :],
                    preferred_element_type=jnp.float32,
                )
                out_ref[pl.ds(my_pos * m_per, m_per), :] = _gelu(y)
            else:
                y = jnp.dot(
                    comm_ref[s, :, :], w_ref[:, :],
                    preferred_element_type=jnp.float32,
                )
                out_ref[pl.ds(tbl_ref[3 + (h - 1)] * m_per, m_per), :] = _gelu(y)
            rdma.wait()

        s_last = (N_DEV - 1) % 2
        y = jnp.dot(
            comm_ref[s_last, :, :], w_ref[:, :],
            preferred_element_type=jnp.float32,
        )
        out_ref[pl.ds(tbl_ref[3 + (N_DEV - 2)] * m_per, m_per), :] = _gelu(y)

    return pl.pallas_call(
        body,
        out_shape=jax.ShapeDtypeStruct((N_DEV * m_per, n_per), jnp.float32),
        in_specs=[
            pl.BlockSpec(memory_space=pltpu.SMEM),
            pl.BlockSpec(memory_space=pltpu.VMEM),
            pl.BlockSpec(memory_space=pltpu.VMEM),
        ],
        out_specs=pl.BlockSpec(memory_space=pltpu.VMEM),
        scratch_shapes=[
            pltpu.VMEM((2, m_per, k_dim), x.dtype),
            pltpu.SemaphoreType.DMA((2,)),
            pltpu.SemaphoreType.DMA((2,)),
        ],
        compiler_params=pltpu.CompilerParams(collective_id=0),
    )(tbl, x, w_mat)


# baseline (device time: 1416517 ns/iter reference)
import jax
import jax.numpy as jnp
from jax import lax
from jax.experimental import pallas as pl
from jax.experimental.pallas import tpu as pltpu

N_DEV = 16

_RING = [0, 4, 8, 12, 13, 9, 5, 1, 2, 6, 10, 14, 15, 11, 7, 3]
_INV = [0] * N_DEV
for _r, _k in enumerate(_RING):
    _INV[_k] = _r
_NEXT = [_RING[(_INV[k] + 1) % N_DEV] for k in range(N_DEV)]
_PREV = [_RING[(_INV[k] - 1) % N_DEV] for k in range(N_DEV)]
_ORIGIN = [
    [_RING[(_INV[k] - (h + 1)) % N_DEV] for k in range(N_DEV)]
    for h in range(N_DEV - 1)
]


def _gelu(y):
    c = 0.7978845608028654
    return 0.5 * y * (1.0 + jnp.tanh(c * (y + 0.044715 * y * y * y)))


def kernel(x, w_mat):
    x = x.astype(jnp.bfloat16)
    w_mat = w_mat.astype(jnp.bfloat16)
    m_per, k_dim = x.shape
    _, n_per = w_mat.shape

    my = lax.axis_index("i")
    nxt = jnp.asarray(_NEXT, dtype=jnp.int32)[my]
    prv = jnp.asarray(_PREV, dtype=jnp.int32)[my]
    origins = jnp.asarray(_ORIGIN, dtype=jnp.int32)[:, my]
    tbl = jnp.concatenate(
        [jnp.stack([my.astype(jnp.int32), nxt, prv]), origins]
    )

    def body(tbl_ref, x_ref, w_ref, out_ref, comm_ref, send_sems, recv_sems):
        my_pos = tbl_ref[0]
        right = tbl_ref[1]
        left = tbl_ref[2]

        barrier = pltpu.get_barrier_semaphore()
        for nbr in (left, right):
            pl.semaphore_signal(
                barrier, inc=1, device_id=(nbr,),
                device_id_type=pl.DeviceIdType.MESH,
            )
        pl.semaphore_wait(barrier, 2)

        comm_ref[0, :, :] = x_ref[:, :]

        for h in range(N_DEV - 1):
            s = h % 2
            r = (h + 1) % 2
            rdma = pltpu.make_async_remote_copy(
                src_ref=comm_ref.at[s],
                dst_ref=comm_ref.at[r],
                send_sem=send_sems.at[s],
                recv_sem=recv_sems.at[r],
                device_id=(right,),
                device_id_type=pl.DeviceIdType.MESH,
            )
            rdma.start()
            if h == 0:
                y = jnp.dot(
                    x_ref[:, :], w_ref[:, :],
                    preferred_element_type=jnp.float32,
                )
                out_ref[pl.ds(my_pos * m_per, m_per), :] = _gelu(y)
            else:
                y = jnp.dot(
                    comm_ref[s, :, :], w_ref[:, :],
                    preferred_element_type=jnp.float32,
                )
                out_ref[pl.ds(tbl_ref[3 + (h - 1)] * m_per, m_per), :] = _gelu(y)
            rdma.wait()

        s_last = (N_DEV - 1) % 2
        y = jnp.dot(
            comm_ref[s_last, :, :], w_ref[:, :],
            preferred_element_type=jnp.float32,
        )
        out_ref[pl.ds(tbl_ref[3 + (N_DEV - 2)] * m_per, m_per), :] = _gelu(y)

    return pl.pallas_call(
        body,
        out_shape=jax.ShapeDtypeStruct((N_DEV * m_per, n_per), jnp.float32),
        in_specs=[
            pl.BlockSpec(memory_space=pltpu.SMEM),
            pl.BlockSpec(memory_space=pltpu.VMEM),
            pl.BlockSpec(memory_space=pltpu.VMEM),
        ],
        out_specs=pl.BlockSpec(memory_space=pltpu.VMEM),
        scratch_shapes=[
            pltpu.VMEM((2, m_per, k_dim), x.dtype),
            pltpu.SemaphoreType.DMA((2,)),
            pltpu.SemaphoreType.DMA((2,)),
        ],
        compiler_params=pltpu.CompilerParams(
            collective_id=0, vmem_limit_bytes=100 * 1024 * 1024
        ),
    )(tbl, x, w_mat)


# device time: 741581 ns/iter; 1.9101x vs baseline; 1.9101x over previous
import jax
import jax.numpy as jnp
from jax import lax
from jax.experimental import pallas as pl
from jax.experimental.pallas import tpu as pltpu

N_DEV = 16

_RING = [0, 4, 8, 12, 13, 9, 5, 1, 2, 6, 10, 14, 15, 11, 7, 3]
_INV = [0] * N_DEV
for _r, _k in enumerate(_RING):
    _INV[_k] = _r
_NEXT = [_RING[(_INV[k] + 1) % N_DEV] for k in range(N_DEV)]
_PREV = [_RING[(_INV[k] - 1) % N_DEV] for k in range(N_DEV)]
_ORIG_CW = [
    [_RING[(_INV[k] - (h + 1)) % N_DEV] for k in range(N_DEV)]
    for h in range(N_DEV - 1)
]
_ORIG_CCW = [
    [_RING[(_INV[k] + (h + 1)) % N_DEV] for k in range(N_DEV)]
    for h in range(N_DEV - 1)
]


def _gelu(y):
    c = 0.7978845608028654
    return 0.5 * y * (1.0 + jnp.tanh(c * (y + 0.044715 * y * y * y)))


def kernel(x, w_mat):
    x = x.astype(jnp.bfloat16)
    w_mat = w_mat.astype(jnp.bfloat16)
    m_per, k_dim = x.shape
    _, n_per = w_mat.shape
    m_half = m_per // 2

    my = lax.axis_index("i")
    nxt = jnp.asarray(_NEXT, dtype=jnp.int32)[my]
    prv = jnp.asarray(_PREV, dtype=jnp.int32)[my]
    orig_cw = jnp.asarray(_ORIG_CW, dtype=jnp.int32)[:, my]
    orig_ccw = jnp.asarray(_ORIG_CCW, dtype=jnp.int32)[:, my]
    tbl = jnp.concatenate(
        [jnp.stack([my.astype(jnp.int32), nxt, prv]), orig_cw, orig_ccw]
    )

    def body(
        tbl_ref, x_ref, w_ref, out_ref,
        cw_ref, ccw_ref, cw_send, cw_recv, ccw_send, ccw_recv,
    ):
        my_pos = tbl_ref[0]
        right = tbl_ref[1]
        left = tbl_ref[2]

        def cw_origin(h):
            return tbl_ref[3 + h]

        def ccw_origin(h):
            return tbl_ref[3 + (N_DEV - 1) + h]

        barrier = pltpu.get_barrier_semaphore()
        for nbr in (left, right):
            pl.semaphore_signal(
                barrier, inc=1, device_id=(nbr,),
                device_id_type=pl.DeviceIdType.MESH,
            )
        pl.semaphore_wait(barrier, 2)

        cw_ref[0, :, :] = x_ref[:m_half, :]
        ccw_ref[0, :, :] = x_ref[m_half:, :]

        for h in range(N_DEV - 1):
            s = h % 2
            r = (h + 1) % 2
            rdma_cw = pltpu.make_async_remote_copy(
                src_ref=cw_ref.at[s],
                dst_ref=cw_ref.at[r],
                send_sem=cw_send.at[s],
                recv_sem=cw_recv.at[r],
                device_id=(right,),
                device_id_type=pl.DeviceIdType.MESH,
            )
            rdma_ccw = pltpu.make_async_remote_copy(
                src_ref=ccw_ref.at[s],
                dst_ref=ccw_ref.at[r],
                send_sem=ccw_send.at[s],
                recv_sem=ccw_recv.at[r],
                device_id=(left,),
                device_id_type=pl.DeviceIdType.MESH,
            )
            rdma_cw.start()
            rdma_ccw.start()
            if h == 0:
                y = jnp.dot(
                    x_ref[:, :], w_ref[:, :],
                    preferred_element_type=jnp.float32,
                )
                out_ref[pl.ds(my_pos * m_per, m_per), :] = _gelu(y)
            else:
                y = jnp.dot(
                    cw_ref[s, :, :], w_ref[:, :],
                    preferred_element_type=jnp.float32,
                )
                out_ref[pl.ds(cw_origin(h - 1) * m_per, m_half), :] = _gelu(y)
                y = jnp.dot(
                    ccw_ref[s, :, :], w_ref[:, :],
                    preferred_element_type=jnp.float32,
                )
                out_ref[pl.ds(ccw_origin(h - 1) * m_per + m_half, m_half), :] = (
                    _gelu(y)
                )
            rdma_cw.wait()
            rdma_ccw.wait()

        s_last = (N_DEV - 1) % 2
        y = jnp.dot(
            cw_ref[s_last, :, :], w_ref[:, :],
            preferred_element_type=jnp.float32,
        )
        out_ref[pl.ds(cw_origin(N_DEV - 2) * m_per, m_half), :] = _gelu(y)
        y = jnp.dot(
            ccw_ref[s_last, :, :], w_ref[:, :],
            preferred_element_type=jnp.float32,
        )
        out_ref[pl.ds(ccw_origin(N_DEV - 2) * m_per + m_half, m_half), :] = (
            _gelu(y)
        )

    return pl.pallas_call(
        body,
        out_shape=jax.ShapeDtypeStruct((N_DEV * m_per, n_per), jnp.float32),
        in_specs=[
            pl.BlockSpec(memory_space=pltpu.SMEM),
            pl.BlockSpec(memory_space=pltpu.VMEM),
            pl.BlockSpec(memory_space=pltpu.VMEM),
        ],
        out_specs=pl.BlockSpec(memory_space=pltpu.VMEM),
        scratch_shapes=[
            pltpu.VMEM((2, m_half, k_dim), x.dtype),
            pltpu.VMEM((2, m_half, k_dim), x.dtype),
            pltpu.SemaphoreType.DMA((2,)),
            pltpu.SemaphoreType.DMA((2,)),
            pltpu.SemaphoreType.DMA((2,)),
            pltpu.SemaphoreType.DMA((2,)),
        ],
        compiler_params=pltpu.CompilerParams(
            collective_id=0, vmem_limit_bytes=100 * 1024 * 1024
        ),
    )(tbl, x, w_mat)


# device time: 718039 ns/iter; 1.9728x vs baseline; 1.0328x over previous
import jax
import jax.numpy as jnp
from jax import lax
from jax.experimental import pallas as pl
from jax.experimental.pallas import tpu as pltpu

N_DEV = 16

_RING = [0, 4, 8, 12, 13, 9, 5, 1, 2, 6, 10, 14, 15, 11, 7, 3]
_INV = [0] * N_DEV
for _r, _k in enumerate(_RING):
    _INV[_k] = _r
_NEXT = [_RING[(_INV[k] + 1) % N_DEV] for k in range(N_DEV)]
_PREV = [_RING[(_INV[k] - 1) % N_DEV] for k in range(N_DEV)]
_ORIG_CW = [
    [_RING[(_INV[k] - (h + 1)) % N_DEV] for k in range(N_DEV)]
    for h in range(N_DEV - 1)
]
_ORIG_CCW = [
    [_RING[(_INV[k] + (h + 1)) % N_DEV] for k in range(N_DEV)]
    for h in range(N_DEV - 1)
]


def _gelu(y):
    c = 0.7978845608028654
    return 0.5 * y * (1.0 + jnp.tanh(c * (y + 0.044715 * y * y * y)))


def kernel(x, w_mat):
    x = x.astype(jnp.bfloat16)
    w_mat = w_mat.astype(jnp.bfloat16)
    m_per, k_dim = x.shape
    _, n_per = w_mat.shape
    m_half = m_per // 2

    my = lax.axis_index("i")
    nxt = jnp.asarray(_NEXT, dtype=jnp.int32)[my]
    prv = jnp.asarray(_PREV, dtype=jnp.int32)[my]
    orig_cw = jnp.asarray(_ORIG_CW, dtype=jnp.int32)[:, my]
    orig_ccw = jnp.asarray(_ORIG_CCW, dtype=jnp.int32)[:, my]
    tbl = jnp.concatenate(
        [jnp.stack([my.astype(jnp.int32), nxt, prv]), orig_cw, orig_ccw]
    )

    SUBS = 2
    sub_m = m_half // SUBS

    def body(
        tbl_ref, x_ref, w_ref, out_ref,
        cw_ref, ccw_ref, cw_send, cw_recv, ccw_send, ccw_recv,
    ):
        my_pos = tbl_ref[0]
        right = tbl_ref[1]
        left = tbl_ref[2]

        def cw_origin(h):
            return tbl_ref[3 + h]

        def ccw_origin(h):
            return tbl_ref[3 + (N_DEV - 1) + h]

        barrier = pltpu.get_barrier_semaphore()
        for nbr in (left, right):
            pl.semaphore_signal(
                barrier, inc=1, device_id=(nbr,),
                device_id_type=pl.DeviceIdType.MESH,
            )
        pl.semaphore_wait(barrier, 2)

        cw_ref[0, :, :] = x_ref[:m_half, :]
        ccw_ref[0, :, :] = x_ref[m_half:, :]

        dirs = (
            ("cw", cw_ref, cw_send, cw_recv, right),
            ("ccw", ccw_ref, ccw_send, ccw_recv, left),
        )
        descs = {}

        for h in range(N_DEV - 1):
            s = h % 2
            r = (h + 1) % 2
            for name, ref, send, recv, dev in dirs:
                for j in range(SUBS):
                    if h >= 2:
                        descs[(h - 2, name, j)].wait_send()
                    if h >= 1:
                        descs[(h - 1, name, j)].wait_recv()
                    d = pltpu.make_async_remote_copy(
                        src_ref=ref.at[s, pl.ds(j * sub_m, sub_m)],
                        dst_ref=ref.at[r, pl.ds(j * sub_m, sub_m)],
                        send_sem=send.at[s, j],
                        recv_sem=recv.at[r, j],
                        device_id=(dev,),
                        device_id_type=pl.DeviceIdType.MESH,
                    )
                    descs[(h, name, j)] = d
                    d.start()
            if h == 0:
                y = jnp.dot(
                    x_ref[:, :], w_ref[:, :],
                    preferred_element_type=jnp.float32,
                )
                out_ref[pl.ds(my_pos * m_per, m_per), :] = _gelu(y)
            else:
                y = jnp.dot(
                    cw_ref[s, :, :], w_ref[:, :],
                    preferred_element_type=jnp.float32,
                )
                out_ref[pl.ds(cw_origin(h - 1) * m_per, m_half), :] = _gelu(y)
                y = jnp.dot(
                    ccw_ref[s, :, :], w_ref[:, :],
                    preferred_element_type=jnp.float32,
                )
                out_ref[pl.ds(ccw_origin(h - 1) * m_per + m_half, m_half), :] = (
                    _gelu(y)
                )

        for name, ref, send, recv, dev in dirs:
            for j in range(SUBS):
                descs[(N_DEV - 3, name, j)].wait_send()
                descs[(N_DEV - 2, name, j)].wait_send()
                descs[(N_DEV - 2, name, j)].wait_recv()

        s_last = (N_DEV - 1) % 2
        y = jnp.dot(
            cw_ref[s_last, :, :], w_ref[:, :],
            preferred_element_type=jnp.float32,
        )
        out_ref[pl.ds(cw_origin(N_DEV - 2) * m_per, m_half), :] = _gelu(y)
        y = jnp.dot(
            ccw_ref[s_last, :, :], w_ref[:, :],
            preferred_element_type=jnp.float32,
        )
        out_ref[pl.ds(ccw_origin(N_DEV - 2) * m_per + m_half, m_half), :] = (
            _gelu(y)
        )

    return pl.pallas_call(
        body,
        out_shape=jax.ShapeDtypeStruct((N_DEV * m_per, n_per), jnp.float32),
        in_specs=[
            pl.BlockSpec(memory_space=pltpu.SMEM),
            pl.BlockSpec(memory_space=pltpu.VMEM),
            pl.BlockSpec(memory_space=pltpu.VMEM),
        ],
        out_specs=pl.BlockSpec(memory_space=pltpu.VMEM),
        scratch_shapes=[
            pltpu.VMEM((2, m_half, k_dim), x.dtype),
            pltpu.VMEM((2, m_half, k_dim), x.dtype),
            pltpu.SemaphoreType.DMA((2, SUBS)),
            pltpu.SemaphoreType.DMA((2, SUBS)),
            pltpu.SemaphoreType.DMA((2, SUBS)),
            pltpu.SemaphoreType.DMA((2, SUBS)),
        ],
        compiler_params=pltpu.CompilerParams(
            collective_id=0, vmem_limit_bytes=100 * 1024 * 1024
        ),
    )(tbl, x, w_mat)


# device time: 707130 ns/iter; 2.0032x vs baseline; 1.0154x over previous
import jax
import jax.numpy as jnp
from jax import lax
from jax.experimental import pallas as pl
from jax.experimental.pallas import tpu as pltpu

N_DEV = 16

_RING = [0, 4, 8, 12, 13, 9, 5, 1, 2, 6, 10, 14, 15, 11, 7, 3]
_INV = [0] * N_DEV
for _r, _k in enumerate(_RING):
    _INV[_k] = _r
_NEXT = [_RING[(_INV[k] + 1) % N_DEV] for k in range(N_DEV)]
_PREV = [_RING[(_INV[k] - 1) % N_DEV] for k in range(N_DEV)]
_ORIG_CW = [
    [_RING[(_INV[k] - (h + 1)) % N_DEV] for k in range(N_DEV)]
    for h in range(N_DEV - 1)
]
_ORIG_CCW = [
    [_RING[(_INV[k] + (h + 1)) % N_DEV] for k in range(N_DEV)]
    for h in range(N_DEV - 1)
]


def _gelu(y):
    c = 0.7978845608028654
    return 0.5 * y * (1.0 + jnp.tanh(c * (y + 0.044715 * y * y * y)))


def kernel(x, w_mat):
    m_per, k_dim = x.shape
    _, n_per = w_mat.shape
    m_half = m_per // 2

    my = lax.axis_index("i")
    nxt = jnp.asarray(_NEXT, dtype=jnp.int32)[my]
    prv = jnp.asarray(_PREV, dtype=jnp.int32)[my]
    orig_cw = jnp.asarray(_ORIG_CW, dtype=jnp.int32)[:, my]
    orig_ccw = jnp.asarray(_ORIG_CCW, dtype=jnp.int32)[:, my]
    tbl = jnp.concatenate(
        [jnp.stack([my.astype(jnp.int32), nxt, prv]), orig_cw, orig_ccw]
    )

    SUBS = 2
    sub_m = m_half // SUBS

    def body(
        tbl_ref, x_ref, w_ref, out_ref,
        cw_ref, ccw_ref, w_bf_ref, cw_send, cw_recv, ccw_send, ccw_recv,
    ):
        my_pos = tbl_ref[0]
        right = tbl_ref[1]
        left = tbl_ref[2]

        def cw_origin(h):
            return tbl_ref[3 + h]

        def ccw_origin(h):
            return tbl_ref[3 + (N_DEV - 1) + h]

        barrier = pltpu.get_barrier_semaphore()
        for nbr in (left, right):
            pl.semaphore_signal(
                barrier, inc=1, device_id=(nbr,),
                device_id_type=pl.DeviceIdType.MESH,
            )
        pl.semaphore_wait(barrier, 2)

        cw_ref[0, :, :] = x_ref[:m_half, :].astype(jnp.bfloat16)
        ccw_ref[0, :, :] = x_ref[m_half:, :].astype(jnp.bfloat16)

        dirs = (
            ("cw", cw_ref, cw_send, cw_recv, right),
            ("ccw", ccw_ref, ccw_send, ccw_recv, left),
        )
        descs = {}

        for h in range(N_DEV - 1):
            s = h % 2
            r = (h + 1) % 2
            for name, ref, send, recv, dev in dirs:
                for j in range(SUBS):
                    if h >= 2:
                        descs[(h - 2, name, j)].wait_send()
                    if h >= 1:
                        descs[(h - 1, name, j)].wait_recv()
                    d = pltpu.make_async_remote_copy(
                        src_ref=ref.at[s, pl.ds(j * sub_m, sub_m)],
                        dst_ref=ref.at[r, pl.ds(j * sub_m, sub_m)],
                        send_sem=send.at[s, j],
                        recv_sem=recv.at[r, j],
                        device_id=(dev,),
                        device_id_type=pl.DeviceIdType.MESH,
                    )
                    descs[(h, name, j)] = d
                    d.start()
            if h == 0:
                w_bf_ref[:, :] = w_ref[:, :].astype(jnp.bfloat16)
                y = jnp.dot(
                    cw_ref[0, :, :], w_bf_ref[:, :],
                    preferred_element_type=jnp.float32,
                )
                out_ref[pl.ds(my_pos * m_per, m_half), :] = _gelu(y)
                y = jnp.dot(
                    ccw_ref[0, :, :], w_bf_ref[:, :],
                    preferred_element_type=jnp.float32,
                )
                out_ref[pl.ds(my_pos * m_per + m_half, m_half), :] = _gelu(y)
            else:
                y = jnp.dot(
                    cw_ref[s, :, :], w_bf_ref[:, :],
                    preferred_element_type=jnp.float32,
                )
                out_ref[pl.ds(cw_origin(h - 1) * m_per, m_half), :] = _gelu(y)
                y = jnp.dot(
                    ccw_ref[s, :, :], w_bf_ref[:, :],
                    preferred_element_type=jnp.float32,
                )
                out_ref[pl.ds(ccw_origin(h - 1) * m_per + m_half, m_half), :] = (
                    _gelu(y)
                )

        s_last = (N_DEV - 1) % 2
        for name, ref, send, recv, dev in dirs:
            base = (
                cw_origin(N_DEV - 2) * m_per
                if name == "cw"
                else ccw_origin(N_DEV - 2) * m_per + m_half
            )
            for j in range(SUBS):
                descs[(N_DEV - 3, name, j)].wait_send()
                descs[(N_DEV - 2, name, j)].wait_send()
                descs[(N_DEV - 2, name, j)].wait_recv()
                y = jnp.dot(
                    ref[s_last, pl.ds(j * sub_m, sub_m), :], w_bf_ref[:, :],
                    preferred_element_type=jnp.float32,
                )
                out_ref[pl.ds(base + j * sub_m, sub_m), :] = _gelu(y)

    return pl.pallas_call(
        body,
        out_shape=jax.ShapeDtypeStruct((N_DEV * m_per, n_per), jnp.float32),
        in_specs=[
            pl.BlockSpec(memory_space=pltpu.SMEM),
            pl.BlockSpec(memory_space=pltpu.VMEM),
            pl.BlockSpec(memory_space=pltpu.VMEM),
        ],
        out_specs=pl.BlockSpec(memory_space=pltpu.VMEM),
        scratch_shapes=[
            pltpu.VMEM((2, m_half, k_dim), jnp.bfloat16),
            pltpu.VMEM((2, m_half, k_dim), jnp.bfloat16),
            pltpu.VMEM((k_dim, n_per), jnp.bfloat16),
            pltpu.SemaphoreType.DMA((2, SUBS)),
            pltpu.SemaphoreType.DMA((2, SUBS)),
            pltpu.SemaphoreType.DMA((2, SUBS)),
            pltpu.SemaphoreType.DMA((2, SUBS)),
        ],
        compiler_params=pltpu.CompilerParams(
            collective_id=0, vmem_limit_bytes=100 * 1024 * 1024
        ),
    )(tbl, x, w_mat)


# device time: 700033 ns/iter; 2.0235x vs baseline; 1.0101x over previous
import jax
import jax.numpy as jnp
from jax import lax
from jax.experimental import pallas as pl
from jax.experimental.pallas import tpu as pltpu

N_DEV = 16

_RING = [0, 4, 8, 12, 13, 9, 5, 1, 2, 6, 10, 14, 15, 11, 7, 3]
_INV = [0] * N_DEV
for _r, _k in enumerate(_RING):
    _INV[_k] = _r
_NEXT = [_RING[(_INV[k] + 1) % N_DEV] for k in range(N_DEV)]
_PREV = [_RING[(_INV[k] - 1) % N_DEV] for k in range(N_DEV)]
_ORIG_CW = [
    [_RING[(_INV[k] - (h + 1)) % N_DEV] for k in range(N_DEV)]
    for h in range(N_DEV - 1)
]
_ORIG_CCW = [
    [_RING[(_INV[k] + (h + 1)) % N_DEV] for k in range(N_DEV)]
    for h in range(N_DEV - 1)
]


def _gelu(y):
    c = 0.7978845608028654
    return 0.5 * y * (1.0 + jnp.tanh(c * (y + 0.044715 * y * y * y)))


def kernel(x, w_mat):
    m_per, k_dim = x.shape
    _, n_per = w_mat.shape
    m_half = m_per // 2

    my = lax.axis_index("i")
    nxt = jnp.asarray(_NEXT, dtype=jnp.int32)[my]
    prv = jnp.asarray(_PREV, dtype=jnp.int32)[my]
    orig_cw = jnp.asarray(_ORIG_CW, dtype=jnp.int32)[:, my]
    orig_ccw = jnp.asarray(_ORIG_CCW, dtype=jnp.int32)[:, my]
    tbl = jnp.concatenate(
        [jnp.stack([my.astype(jnp.int32), nxt, prv]), orig_cw, orig_ccw]
    )

    SUBS = 2
    sub_m = m_half // SUBS

    def body(
        tbl_ref, x_ref, w_ref, out_ref,
        cw_ref, ccw_ref, w_bf_ref, xtmp_ref, wtmp_ref,
        cw_send, cw_recv, ccw_send, ccw_recv, x_sems, w_sem,
    ):
        my_pos = tbl_ref[0]
        right = tbl_ref[1]
        left = tbl_ref[2]

        def cw_origin(h):
            return tbl_ref[3 + h]

        def ccw_origin(h):
            return tbl_ref[3 + (N_DEV - 1) + h]

        fill_order = [("cw", 0, 0), ("ccw", 0, m_half), ("cw", 1, 0),
                      ("ccw", 1, m_half)]
        xdma = []
        for t, (name, j, base_row) in enumerate(fill_order):
            d = pltpu.make_async_copy(
                x_ref.at[pl.ds(base_row + j * sub_m, sub_m)],
                xtmp_ref.at[t],
                x_sems.at[t],
            )
            d.start()
            xdma.append(d)
        wdma = pltpu.make_async_copy(w_ref, wtmp_ref, w_sem)
        wdma.start()

        barrier = pltpu.get_barrier_semaphore()
        for nbr in (left, right):
            pl.semaphore_signal(
                barrier, inc=1, device_id=(nbr,),
                device_id_type=pl.DeviceIdType.MESH,
            )
        pl.semaphore_wait(barrier, 2)

        dirs = (
            ("cw", cw_ref, cw_send, cw_recv, right),
            ("ccw", ccw_ref, ccw_send, ccw_recv, left),
        )
        by_name = {d[0]: d for d in dirs}
        descs = {}

        for t, (name, j, base_row) in enumerate(fill_order):
            _, ref, send, recv, dev = by_name[name]
            xdma[t].wait()
            ref[0, pl.ds(j * sub_m, sub_m), :] = (
                xtmp_ref[t].astype(jnp.bfloat16)
            )
            d = pltpu.make_async_remote_copy(
                src_ref=ref.at[0, pl.ds(j * sub_m, sub_m)],
                dst_ref=ref.at[1, pl.ds(j * sub_m, sub_m)],
                send_sem=send.at[0, j],
                recv_sem=recv.at[1, j],
                device_id=(dev,),
                device_id_type=pl.DeviceIdType.MESH,
            )
            descs[(0, name, j)] = d
            d.start()

        wdma.wait()
        w_bf_ref[:, :] = wtmp_ref[:, :].astype(jnp.bfloat16)
        y = jnp.dot(
            cw_ref[0, :, :], w_bf_ref[:, :], preferred_element_type=jnp.float32
        )
        out_ref[pl.ds(my_pos * m_per, m_half), :] = _gelu(y)
        y = jnp.dot(
            ccw_ref[0, :, :], w_bf_ref[:, :], preferred_element_type=jnp.float32
        )
        out_ref[pl.ds(my_pos * m_per + m_half, m_half), :] = _gelu(y)

        for h in range(1, N_DEV - 1):
            s = h % 2
            r = (h + 1) % 2
            for name, ref, send, recv, dev in dirs:
                for j in range(SUBS):
                    if h >= 2:
                        descs[(h - 2, name, j)].wait_send()
                    descs[(h - 1, name, j)].wait_recv()
                    d = pltpu.make_async_remote_copy(
                        src_ref=ref.at[s, pl.ds(j * sub_m, sub_m)],
                        dst_ref=ref.at[r, pl.ds(j * sub_m, sub_m)],
                        send_sem=send.at[s, j],
                        recv_sem=recv.at[r, j],
                        device_id=(dev,),
                        device_id_type=pl.DeviceIdType.MESH,
                    )
                    descs[(h, name, j)] = d
                    d.start()
            y = jnp.dot(
                cw_ref[s, :, :], w_bf_ref[:, :],
                preferred_element_type=jnp.float32,
            )
            out_ref[pl.ds(cw_origin(h - 1) * m_per, m_half), :] = _gelu(y)
            y = jnp.dot(
                ccw_ref[s, :, :], w_bf_ref[:, :],
                preferred_element_type=jnp.float32,
            )
            out_ref[pl.ds(ccw_origin(h - 1) * m_per + m_half, m_half), :] = (
                _gelu(y)
            )

        s_last = (N_DEV - 1) % 2
        for name, ref, send, recv, dev in dirs:
            base = (
                cw_origin(N_DEV - 2) * m_per
                if name == "cw"
                else ccw_origin(N_DEV - 2) * m_per + m_half
            )
            for j in range(SUBS):
                descs[(N_DEV - 3, name, j)].wait_send()
                descs[(N_DEV - 2, name, j)].wait_send()
                descs[(N_DEV - 2, name, j)].wait_recv()
                y = jnp.dot(
                    ref[s_last, pl.ds(j * sub_m, sub_m), :], w_bf_ref[:, :],
                    preferred_element_type=jnp.float32,
                )
                out_ref[pl.ds(base + j * sub_m, sub_m), :] = _gelu(y)

    return pl.pallas_call(
        body,
        out_shape=jax.ShapeDtypeStruct((N_DEV * m_per, n_per), jnp.float32),
        in_specs=[
            pl.BlockSpec(memory_space=pltpu.SMEM),
            pl.BlockSpec(memory_space=pl.ANY),
            pl.BlockSpec(memory_space=pl.ANY),
        ],
        out_specs=pl.BlockSpec(memory_space=pltpu.VMEM),
        scratch_shapes=[
            pltpu.VMEM((2, m_half, k_dim), jnp.bfloat16),
            pltpu.VMEM((2, m_half, k_dim), jnp.bfloat16),
            pltpu.VMEM((k_dim, n_per), jnp.bfloat16),
            pltpu.VMEM((2 * SUBS, sub_m, k_dim), jnp.float32),
            pltpu.VMEM((k_dim, n_per), jnp.float32),
            pltpu.SemaphoreType.DMA((2, SUBS)),
            pltpu.SemaphoreType.DMA((2, SUBS)),
            pltpu.SemaphoreType.DMA((2, SUBS)),
            pltpu.SemaphoreType.DMA((2, SUBS)),
            pltpu.SemaphoreType.DMA((2 * SUBS,)),
            pltpu.SemaphoreType.DMA,
        ],
        compiler_params=pltpu.CompilerParams(
            collective_id=0, vmem_limit_bytes=100 * 1024 * 1024
        ),
    )(tbl, x, w_mat)
